# baseline (speedup 1.0000x reference)
import jax
import jax.numpy as jnp
from jax import lax
from jax.experimental import pallas as pl
from jax.experimental.pallas import tpu as pltpu
from jax.experimental.pallas import tpu_sc as plsc

_B = 4096
_NVARS = 128
_NUM_NODES = 256
_NN = _B * _NUM_NODES
_E = 8388608
_NITER = 10
_NC = 2
_NS = 16
_NW = _NC * _NS
_L = 16
_HALF = _NN // 2
_C = 1024
_NB = 4
_EPTW = _E // _NW
_NPCH = _EPTW // _C
_REG = _EPTW + _NB * _C
_XW = _NN // _NS
_AW = _HALF // _NS


def _dyngather(x, idx):
  dn = lax.GatherDimensionNumbers(offset_dims=(), collapsed_slice_dims=(0,),
                                  start_index_map=(0,))
  return lax.gather(x, idx[:, None], dn, (1,),
                    mode=lax.GatherScatterMode.PROMISE_IN_BOUNDS)


def _part_body(src_hbm, dst_hbm, cm_hbm, psrc, pdst, pcm, cnt_hbm, *rest):
  (sv, dv, cv, posv, pad_s, pad_d, pad_f, cntv) = rest
  c = lax.axis_index("c")
  s = lax.axis_index("s")
  t = c * _NS + s

  iot = lax.iota(jnp.int32, _L)
  perm15 = jnp.full((_L,), _L - 1, jnp.int32)
  perms = [jnp.maximum(iot - d, 0) for d in (1, 2, 4, 8)]
  gmasks = [iot >= d for d in (1, 2, 4, 8)]
  zi = jnp.zeros((_L,), jnp.int32)
  zf = jnp.zeros((_L,), jnp.float32)
  nn = jnp.full((_L,), _NN, jnp.int32)
  one = jnp.full((_L,), 1, jnp.int32)
  zero = jnp.full((_L,), 0, jnp.int32)

  @pl.loop(0, _C // _L)
  def _fill(j):
    sl = pl.ds(j * _L, _L)
    pad_s[sl] = zi
    pad_d[sl] = nn
    pad_f[sl] = zf

  def _scatter_pads():
    pltpu.sync_copy(pad_s, psrc.at[posv])
    pltpu.sync_copy(pad_d, pdst.at[posv])
    pltpu.sync_copy(pad_f, pcm.at[posv])

  lo0 = zero + t * _REG
  hi0 = zero + (_NW + t) * _REG
  e_base = t * _EPTW

  @pl.loop(0, _NPCH, init_carry=(lo0, hi0))
  def _chunks(k, carry):
    lo_in, hi_in = carry
    e0 = e_base + k * _C
    pltpu.sync_copy(src_hbm.at[pl.ds(e0, _C)], sv)
    pltpu.sync_copy(dst_hbm.at[pl.ds(e0, _C)], dv)
    pltpu.sync_copy(cm_hbm.at[pl.ds(e0, _C)], cv)

    @pl.loop(0, _C // _L, init_carry=(lo_in, hi_in))
    def _vec(i, car):
      lo, hi = car
      sl = pl.ds(i * _L, _L)
      d_vec = dv[sl]
      m = d_vec < _HALF
      inc = jnp.where(m, one, zero)
      cs = inc
      for pm, gm in zip(perms, gmasks):
        sh = _dyngather(cs, pm)
        cs = cs + jnp.where(gm, sh, zero)
      excl = cs - inc
      n = _dyngather(cs, perm15)
      posv[sl] = jnp.where(m, lo + excl, hi + (iot - excl))
      return lo + n, hi + (_L - n)

    lo_out, hi_out = _vec
    pltpu.sync_copy(sv, psrc.at[posv])
    pltpu.sync_copy(dv, pdst.at[posv])
    pltpu.sync_copy(cv, pcm.at[posv])
    return lo_out, hi_out

  lo_vec, hi_vec = _chunks

  for h, vec in ((0, lo_vec), (1, hi_vec)):
    basev = zero + (h * _NW + t) * _REG
    offv = vec - basev
    nchv = lax.shift_right_logical(offv + (_C - 1), 10)
    topbase = basev + nchv * _C

    @pl.loop(0, _C // _L)
    def _tp(j, _pb=vec):
      posv[pl.ds(j * _L, _L)] = _pb + j * _L + iot

    _scatter_pads()

    for q in range(_NB):

      @pl.loop(0, _C // _L)
      def _tq(j, _tb=topbase, _q=q):
        posv[pl.ds(j * _L, _L)] = _tb + _q * _C + j * _L + iot

      _scatter_pads()

    cntv[pl.ds(h * _L, _L)] = jnp.maximum(nchv, _NB)

  pltpu.sync_copy(cntv.at[pl.ds(0, _L)],
                  cnt_hbm.at[pl.ds(t * _L, _L)])
  pltpu.sync_copy(cntv.at[pl.ds(_L, _L)],
                  cnt_hbm.at[pl.ds((_NW + t) * _L, _L)])


_partition = pl.kernel(
    _part_body,
    out_type=(jax.ShapeDtypeStruct((2 * _NW * _REG,), jnp.int32),
              jax.ShapeDtypeStruct((2 * _NW * _REG,), jnp.int32),
              jax.ShapeDtypeStruct((2 * _NW * _REG,), jnp.float32),
              jax.ShapeDtypeStruct((2 * _NW * _L,), jnp.int32)),
    mesh=plsc.VectorSubcoreMesh(core_axis_name="c", subcore_axis_name="s",
                                num_cores=_NC, num_subcores=_NS),
    scratch_types=[
        pltpu.VMEM((_C,), jnp.int32),
        pltpu.VMEM((_C,), jnp.int32),
        pltpu.VMEM((_C,), jnp.float32),
        pltpu.VMEM((_C,), jnp.int32),
        pltpu.VMEM((_C,), jnp.int32),
        pltpu.VMEM((_C,), jnp.int32),
        pltpu.VMEM((_C,), jnp.float32),
        pltpu.VMEM((2 * _L,), jnp.int32),
    ],
)


def _iter_body(x_hbm, base_hbm, psrc, pdst, pcm, cnt_hbm, par_hbm,
               xout_hbm, *rest):
  srcv = rest[0:_NB]
  dstv = rest[_NB:2 * _NB]
  xsv = rest[2 * _NB:3 * _NB]
  xdv = rest[3 * _NB:4 * _NB]
  msgv = rest[4 * _NB:5 * _NB]
  dlv = rest[5 * _NB:6 * _NB]
  cmv, parv, x_sp, acc_sp, cnt_sp, cnt_sm = rest[6 * _NB:6 * _NB + 6]
  lsem = rest[6 * _NB + 6:6 * _NB + 6 + _NB]
  gsem = rest[6 * _NB + 6 + _NB:6 * _NB + 6 + 2 * _NB]

  c = lax.axis_index("c")
  s = lax.axis_index("s")

  pltpu.sync_copy(x_hbm.at[pl.ds(s * _XW, _XW)], x_sp.at[pl.ds(s * _XW, _XW)])
  pltpu.sync_copy(base_hbm.at[pl.ds(c * _HALF + s * _AW, _AW)],
                  acc_sp.at[pl.ds(s * _AW, _AW)])
  pltpu.sync_copy(par_hbm, parv)

  @pl.when(s == 0)
  def _():
    pltpu.sync_copy(cnt_hbm, cnt_sp)

  plsc.subcore_barrier()

  g0 = c * _NW + 2 * s
  pltpu.sync_copy(cnt_sp.at[pl.ds(g0 * _L, 2 * _L)], cnt_sm)
  nch0 = cnt_sm[0]
  nch1 = cnt_sm[_L]
  tot = nch0 + nch1
  base0 = g0 * _REG
  base1 = (g0 + 1) * _REG

  w0 = parv[pl.ds(0, _L)]
  w1 = parv[pl.ds(16, _L)]
  w2 = parv[pl.ds(32, _L)]
  bb = parv[pl.ds(48, _L)]
  pen = parv[pl.ds(64, _L)]
  scal = parv[pl.ds(80, _L)]

  def _e0(k):
    return jnp.where(k < nch0, base0 + k * _C, base1 + (k - nch0) * _C)

  def _lin_start(b, k):
    e0 = _e0(k)
    pltpu.async_copy(psrc.at[pl.ds(e0, _C)], srcv[b], lsem[b])
    pltpu.async_copy(pdst.at[pl.ds(e0, _C)], dstv[b], lsem[b])
    pltpu.async_copy(pcm.at[pl.ds(e0, _C)], cmv.at[b], lsem[b])

  def _lin_wait(b, k):
    e0 = _e0(k)
    pltpu.make_async_copy(psrc.at[pl.ds(e0, _C)], srcv[b], lsem[b]).wait()
    pltpu.make_async_copy(pdst.at[pl.ds(e0, _C)], dstv[b], lsem[b]).wait()
    pltpu.make_async_copy(pcm.at[pl.ds(e0, _C)], cmv.at[b], lsem[b]).wait()

  def _gat_start(b):
    pltpu.async_copy(x_sp.at[srcv[b]], xsv[b], gsem[b])
    pltpu.async_copy(x_sp.at[dstv[b]], xdv[b], gsem[b])

  def _gat_wait(b):
    pltpu.make_async_copy(x_sp.at[srcv[b]], xsv[b], gsem[b]).wait()
    pltpu.make_async_copy(x_sp.at[dstv[b]], xdv[b], gsem[b]).wait()

  for b in range(_NB - 1):
    _lin_start(b, b)
  _lin_wait(0, 0)
  _gat_start(0)

  @pl.loop(0, tot, step=_NB)
  def _chunks(k):
    for b in range(_NB):
      kk = k + b

      @pl.when(kk < tot)
      def _():
        bn = (b + 1) % _NB

        @pl.when(kk + 1 < tot)
        def _():
          _lin_wait(bn, kk + 1)
          _gat_start(bn)

        _gat_wait(b)

        @pl.loop(0, _C // _L, unroll=4)
        def _vec(i):
          sl = pl.ds(i * _L, _L)
          xs = xsv[b][sl]
          xd = xdv[b][sl]
          cmx = cmv[b, sl]
          dd = dstv[b][sl]
          r = xs * w0 + xd * w1 + cmx * w2 + bb
          r = jnp.maximum(r, r * jnp.float32(0.01))
          r = r + cmx * pen
          xx = jnp.clip(-r, jnp.float32(-87.33), jnp.float32(88.72))
          tt = xx * jnp.float32(1.44269504088896341)
          km = (tt + jnp.float32(12582912.0)) - jnp.float32(12582912.0)
          f = xx - km * jnp.float32(0.693359375)
          f = f - km * jnp.float32(-2.12194440e-4)
          z = f * f
          pp = jnp.float32(1.9875691500e-4)
          pp = pp * f + jnp.float32(1.3981999507e-3)
          pp = pp * f + jnp.float32(8.3334519073e-3)
          pp = pp * f + jnp.float32(4.1665795894e-2)
          pp = pp * f + jnp.float32(1.6666665459e-1)
          pp = pp * f + jnp.float32(5.0000001201e-1)
          ee = z * pp + f + jnp.float32(1.0)
          ki = km.astype(jnp.int32)
          sc2 = lax.bitcast_convert_type(
              lax.shift_left(ki + 127, 23), jnp.float32)
          ee = ee * sc2
          a = jnp.float32(1.0) / (jnp.float32(1.0) + ee)
          m = xs * a * scal
          ok = lax.shift_right_logical(dd, 19) == c
          msgv[b][sl] = jnp.where(ok, m, jnp.float32(0.0))
          dlv[b][sl] = lax.bitwise_and(dd, _HALF - 1)

        pltpu.sync_copy(msgv[b], acc_sp.at[dlv[b]], add=True)

        @pl.when(kk + _NB - 1 < tot)
        def _():
          _lin_start((b + _NB - 1) % _NB, kk + _NB - 1)

  plsc.subcore_barrier()
  pltpu.sync_copy(acc_sp.at[pl.ds(s * _AW, _AW)],
                  xout_hbm.at[pl.ds(c * _HALF + s * _AW, _AW)])


_decode_iter = pl.kernel(
    _iter_body,
    out_type=jax.ShapeDtypeStruct((_NN,), jnp.float32),
    mesh=plsc.VectorSubcoreMesh(core_axis_name="c", subcore_axis_name="s",
                                num_cores=_NC, num_subcores=_NS),
    scratch_types=(
        [pltpu.VMEM((_C,), jnp.int32)] * _NB
        + [pltpu.VMEM((_C,), jnp.int32)] * _NB
        + [pltpu.VMEM((_C,), jnp.float32)] * _NB
        + [pltpu.VMEM((_C,), jnp.float32)] * _NB
        + [pltpu.VMEM((_C,), jnp.float32)] * _NB
        + [pltpu.VMEM((_C,), jnp.int32)] * _NB
        + [pltpu.VMEM((_NB, _C), jnp.float32),
           pltpu.VMEM((6 * _L,), jnp.float32),
           pltpu.VMEM_SHARED((_NN,), jnp.float32),
           pltpu.VMEM_SHARED((_HALF,), jnp.float32),
           pltpu.VMEM_SHARED((2 * _NW * _L,), jnp.int32),
           pltpu.SMEM((2 * _L,), jnp.int32)]
        + [pltpu.SemaphoreType.DMA] * (2 * _NB)
    ),
)


def kernel(initial_llrs, edge_index, cycle_mask, att_W, att_b,
           min_sum_scaler, cycle_penalty):
  base = jnp.concatenate(
      [initial_llrs,
       jnp.zeros((_B, _NUM_NODES - _NVARS), initial_llrs.dtype)],
      axis=1).reshape(-1)
  src = edge_index[0]
  dst = edge_index[1]
  p = jnp.stack([att_W[:, 0, 0], att_W[:, 0, 1], att_W[:, 0, 2],
                 att_b[:, 0], cycle_penalty[:, 0], min_sum_scaler[:, 0]],
                axis=1)
  params = jnp.broadcast_to(p[:, :, None],
                            (_NITER, 6, _L)).reshape(_NITER, 6 * _L)
  params = params.astype(jnp.float32)

  psrc, pdst, pcm, cnts = _partition(src, dst, cycle_mask)

  x = base
  outs = []
  for i in range(_NITER):
    x = _decode_iter(x, base, psrc, pdst, pcm, cnts, params[i])
    outs.append(x.reshape(_B, _NUM_NODES)[:, :_NVARS])
  return tuple(outs)


# speedup vs baseline: 4.6909x; 4.6909x over previous
import jax
import jax.numpy as jnp
from jax import lax
from jax.experimental import pallas as pl
from jax.experimental.pallas import tpu as pltpu
from jax.experimental.pallas import tpu_sc as plsc

_B = 4096
_NVARS = 128
_NUM_NODES = 256
_NN = _B * _NUM_NODES
_E = 8388608
_NITER = 10
_NC = 2
_NS = 16
_L = 16
_HALF = _NN // 2
_CHUNK = 1024
_EPT = _E // _NS
_NCHUNKS = _EPT // _CHUNK
_XW = _NN // _NS
_AW = _HALF // _NS
_NB = 4


def _iter_body(x_hbm, base_hbm, src_hbm, dst_hbm, cm_hbm, par_hbm,
               xout_hbm, *rest):
  srcv = rest[0:_NB]
  dstv = rest[_NB:2 * _NB]
  xsv = rest[2 * _NB:3 * _NB]
  xdv = rest[3 * _NB:4 * _NB]
  msgv = rest[4 * _NB:5 * _NB]
  dlv = rest[5 * _NB:6 * _NB]
  cmv, parv, x_sp, acc_sp = rest[6 * _NB:6 * _NB + 4]
  lsem = rest[6 * _NB + 4:6 * _NB + 4 + _NB]
  gsem = rest[6 * _NB + 4 + _NB:6 * _NB + 4 + 2 * _NB]

  c = lax.axis_index("c")
  s = lax.axis_index("s")

  pltpu.sync_copy(x_hbm.at[pl.ds(s * _XW, _XW)], x_sp.at[pl.ds(s * _XW, _XW)])
  pltpu.sync_copy(base_hbm.at[pl.ds(c * _HALF + s * _AW, _AW)],
                  acc_sp.at[pl.ds(s * _AW, _AW)])
  pltpu.sync_copy(par_hbm, parv)
  plsc.subcore_barrier()

  w0 = parv[pl.ds(0, _L)]
  w1 = parv[pl.ds(16, _L)]
  w2 = parv[pl.ds(32, _L)]
  bb = parv[pl.ds(48, _L)]
  pen = parv[pl.ds(64, _L)]
  scal = parv[pl.ds(80, _L)]

  def _e0(k):
    return s * _EPT + k * _CHUNK

  def _lin_start(b, k):
    e0 = _e0(k)
    pltpu.async_copy(src_hbm.at[pl.ds(e0, _CHUNK)], srcv[b], lsem[b])
    pltpu.async_copy(dst_hbm.at[pl.ds(e0, _CHUNK)], dstv[b], lsem[b])
    pltpu.async_copy(cm_hbm.at[pl.ds(e0, _CHUNK)], cmv.at[b], lsem[b])

  def _lin_wait(b, k):
    e0 = _e0(k)
    pltpu.make_async_copy(src_hbm.at[pl.ds(e0, _CHUNK)], srcv[b],
                          lsem[b]).wait()
    pltpu.make_async_copy(dst_hbm.at[pl.ds(e0, _CHUNK)], dstv[b],
                          lsem[b]).wait()
    pltpu.make_async_copy(cm_hbm.at[pl.ds(e0, _CHUNK)], cmv.at[b],
                          lsem[b]).wait()

  def _gat_start(b):
    pltpu.async_copy(x_sp.at[srcv[b]], xsv[b], gsem[b])
    pltpu.async_copy(x_sp.at[dstv[b]], xdv[b], gsem[b])

  def _gat_wait(b):
    pltpu.make_async_copy(x_sp.at[srcv[b]], xsv[b], gsem[b]).wait()
    pltpu.make_async_copy(x_sp.at[dstv[b]], xdv[b], gsem[b]).wait()

  for b in range(_NB - 1):
    _lin_start(b, b)
  _lin_wait(0, 0)
  _gat_start(0)

  @pl.loop(0, _NCHUNKS, step=_NB)
  def _chunks(k):
    for b in range(_NB):
      kk = k + b
      bn = (b + 1) % _NB

      @pl.when(kk + 1 < _NCHUNKS)
      def _():
        _lin_wait(bn, kk + 1)
        _gat_start(bn)

      _gat_wait(b)

      @pl.loop(0, _CHUNK // _L, unroll=4)
      def _vec(i):
        sl = pl.ds(i * _L, _L)
        xs = xsv[b][sl]
        xd = xdv[b][sl]
        cmx = cmv[b, sl]
        dd = dstv[b][sl]
        r = xs * w0 + xd * w1 + cmx * w2 + bb
        r = jnp.maximum(r, r * jnp.float32(0.01))
        r = r + cmx * pen
        a = jnp.float32(1.0) / (jnp.float32(1.0) + jnp.exp(-r))
        m = xs * a * scal
        ok = lax.shift_right_logical(dd, 19) == c
        msgv[b][sl] = jnp.where(ok, m, jnp.float32(0.0))
        dlv[b][sl] = lax.bitwise_and(dd, _HALF - 1)

      pltpu.sync_copy(msgv[b], acc_sp.at[dlv[b]], add=True)

      @pl.when(kk + _NB - 1 < _NCHUNKS)
      def _():
        _lin_start((b + _NB - 1) % _NB, kk + _NB - 1)

  plsc.subcore_barrier()
  pltpu.sync_copy(acc_sp.at[pl.ds(s * _AW, _AW)],
                  xout_hbm.at[pl.ds(c * _HALF + s * _AW, _AW)])


_decode_iter = pl.kernel(
    _iter_body,
    out_type=jax.ShapeDtypeStruct((_NN,), jnp.float32),
    mesh=plsc.VectorSubcoreMesh(core_axis_name="c", subcore_axis_name="s",
                                num_cores=_NC, num_subcores=_NS),
    scratch_types=(
        [pltpu.VMEM((_CHUNK,), jnp.int32)] * _NB
        + [pltpu.VMEM((_CHUNK,), jnp.int32)] * _NB
        + [pltpu.VMEM((_CHUNK,), jnp.float32)] * _NB
        + [pltpu.VMEM((_CHUNK,), jnp.float32)] * _NB
        + [pltpu.VMEM((_CHUNK,), jnp.float32)] * _NB
        + [pltpu.VMEM((_CHUNK,), jnp.int32)] * _NB
        + [pltpu.VMEM((_NB, _CHUNK), jnp.float32),
           pltpu.VMEM((6 * _L,), jnp.float32),
           pltpu.VMEM_SHARED((_NN,), jnp.float32),
           pltpu.VMEM_SHARED((_HALF,), jnp.float32)]
        + [pltpu.SemaphoreType.DMA] * (2 * _NB)
    ),
)


def kernel(initial_llrs, edge_index, cycle_mask, att_W, att_b,
           min_sum_scaler, cycle_penalty):
  base = jnp.concatenate(
      [initial_llrs,
       jnp.zeros((_B, _NUM_NODES - _NVARS), initial_llrs.dtype)],
      axis=1).reshape(-1)
  src = edge_index[0]
  dst = edge_index[1]
  p = jnp.stack([att_W[:, 0, 0], att_W[:, 0, 1], att_W[:, 0, 2],
                 att_b[:, 0], cycle_penalty[:, 0], min_sum_scaler[:, 0]],
                axis=1)
  params = jnp.broadcast_to(p[:, :, None],
                            (_NITER, 6, _L)).reshape(_NITER, 6 * _L)
  params = params.astype(jnp.float32)
  x = base
  outs = []
  for i in range(_NITER):
    x = _decode_iter(x, base, src, dst, cycle_mask, params[i])
    outs.append(x.reshape(_B, _NUM_NODES)[:, :_NVARS])
  return tuple(outs)
